# baseline (speedup 1.0000x reference)
import functools

import jax
import jax.numpy as jnp
from jax import lax
from jax.experimental import pallas as pl
from jax.experimental.pallas import tpu as pltpu
from jax.experimental.pallas import tpu_sc as plsc

B, F, V, D = 16384, 26, 100000, 16
T = F * D
NW = 32
CH = 1024
NCB = B // CH

FCH = 2
NKC = F // FCH

BT = 1024


def _sc_gather_body(tableC, xT, out, idx_v, rows_v, gsem):
    wid = lax.axis_index("s") * 2 + lax.axis_index("c")
    f = wid >> 4
    c = wid & (NCB - 1)
    pltpu.sync_copy(xT.at[f, pl.ds(c * CH, CH)], idx_v)
    pltpu.async_copy(tableC.at[f].at[idx_v], rows_v, gsem).wait()
    pltpu.sync_copy(rows_v, out.at[pl.ds(c * CH, CH), pl.ds(f * D, D)])


@jax.jit
def _sc_gather(tableC, xT):
    mesh = plsc.VectorSubcoreMesh(core_axis_name="c", subcore_axis_name="s")
    k = functools.partial(
        pl.kernel,
        mesh=mesh,
        out_type=jax.ShapeDtypeStruct((B, FCH * D), jnp.float32),
        name="sc_gather",
        scratch_types=[
            pltpu.VMEM((CH,), jnp.int32),
            pltpu.VMEM((CH, D), jnp.float32),
            pltpu.SemaphoreType.DMA,
        ],
        compiler_params=pltpu.CompilerParams(use_tc_tiling_on_sc=False),
    )(_sc_gather_body)
    return k(tableC, xT)


def _tc_body(h_ref, W1_ref, b1_ref, W2_ref, b2_ref, Wfm_ref, Wsm_ref,
             consts_ref, out_ref):
    h = h_ref[...]
    dn = (((1,), (0,)), ((), ()))
    m1 = jnp.maximum(
        lax.dot_general(h, W1_ref[...], dn,
                        preferred_element_type=jnp.float32) + b1_ref[...], 0.0)
    ss = lax.dot_general(h, Wsm_ref[...], dn,
                         preferred_element_type=jnp.float32)
    cv = consts_ref[...]
    probs = []
    for t in range(3):
        m1t = m1[:, t * 256:(t + 1) * 256]
        m2 = jnp.maximum(
            lax.dot_general(m1t, W2_ref[t], dn,
                            preferred_element_type=jnp.float32)
            + b2_ref[t][None, :], 0.0)
        s = jnp.sum(m2 * Wfm_ref[t][None, :], axis=1, keepdims=True)
        a = ss[:, 2 * t:2 * t + 1]
        c = ss[:, 2 * t + 1:2 * t + 2]
        logit = a * c + c + s + cv[0, t]
        probs.append(jax.nn.sigmoid(logit))
    cvr, cf, ctr = probs
    ctcvr = cvr * ctr
    res = jnp.concatenate([cvr, cf, ctr, ctcvr], axis=1)
    out_ref[...] = jnp.clip(res, 1e-15, 1.0 - 1e-15)


@jax.jit
def _tc_dense(h, W1all, b1all, W2all, b2all, Wfm, Wsm, consts):
    full = lambda shape: pl.BlockSpec(shape, lambda i: (0,) * len(shape))
    return pl.pallas_call(
        _tc_body,
        grid=(B // BT,),
        in_specs=[
            pl.BlockSpec((BT, T), lambda i: (i, 0)),
            full((T, 768)),
            full((1, 768)),
            full((3, 256, 128)),
            full((3, 128)),
            full((3, 128)),
            full((T, 8)),
            full((1, 8)),
        ],
        out_specs=pl.BlockSpec((BT, 4), lambda i: (i, 0)),
        out_shape=jax.ShapeDtypeStruct((B, 4), jnp.float32),
        compiler_params=pltpu.CompilerParams(
            dimension_semantics=("parallel",)),
    )(h, W1all, b1all, W2all, b2all, Wfm, Wsm, consts)


def kernel(x, emb_tables,
           cvr_wc, cvr_bc, cvr_W1, cvr_b1, cvr_W2, cvr_b2, cvr_Wf, cvr_bf,
           cf_wc, cf_bc, cf_W1, cf_b1, cf_W2, cf_b2, cf_Wf, cf_bf,
           ctr_wc, ctr_bc, ctr_W1, ctr_b1, ctr_W2, ctr_b2, ctr_Wf, ctr_bf):
    xT = x.astype(jnp.int32).T

    W1all = jnp.concatenate([cvr_W1, cf_W1, ctr_W1], axis=1)
    b1all = jnp.concatenate([cvr_b1, cf_b1, ctr_b1]).reshape(1, 768)
    W2all = jnp.stack([cvr_W2, cf_W2, ctr_W2])
    b2all = jnp.stack([cvr_b2, cf_b2, ctr_b2])
    Wfm = jnp.stack([cvr_Wf[T:, 0], cf_Wf[T:, 0], ctr_Wf[T:, 0]])
    zcol = jnp.zeros((T, 1), jnp.float32)
    Wsm = jnp.concatenate(
        [cvr_wc, cvr_Wf[:T], cf_wc, cf_Wf[:T], ctr_wc, ctr_Wf[:T],
         zcol, zcol], axis=1)
    consts = jnp.stack(
        [jnp.dot(cvr_bc, cvr_Wf[:T, 0]) + cvr_bf[0],
         jnp.dot(cf_bc, cf_Wf[:T, 0]) + cf_bf[0],
         jnp.dot(ctr_bc, ctr_Wf[:T, 0]) + ctr_bf[0],
         jnp.float32(0), jnp.float32(0), jnp.float32(0),
         jnp.float32(0), jnp.float32(0)]).reshape(1, 8)

    h_parts = [
        _sc_gather(emb_tables[k * FCH:(k + 1) * FCH],
                   xT[k * FCH:(k + 1) * FCH])
        for k in range(NKC)
    ]
    h = jnp.concatenate(h_parts, axis=1)

    return _tc_dense(h, W1all, b1all, W2all, b2all, Wfm, Wsm, consts)


# speedup vs baseline: 3.3129x; 3.3129x over previous
import functools

import jax
import jax.numpy as jnp
from jax import lax
from jax.experimental import pallas as pl
from jax.experimental.pallas import tpu as pltpu
from jax.experimental.pallas import tpu_sc as plsc

B, F, V, D = 16384, 26, 100000, 16
T = F * D
NW = 32
CH = 1024
NCB = B // CH

FCH = 2
NKC = F // FCH

BT = 1024


def _sc_gather_body(tableT, xT, out, idx_v, rows_v, gsem):
    wid = lax.axis_index("s") * 2 + lax.axis_index("c")
    f = wid >> 4
    c = wid & (NCB - 1)
    pltpu.sync_copy(xT.at[f, pl.ds(c * CH, CH)], idx_v)
    copies = [
        pltpu.async_copy(tableT.at[f, d].at[idx_v], rows_v.at[d], gsem)
        for d in range(D)
    ]
    for cp in copies:
        cp.wait()
    pltpu.sync_copy(rows_v, out.at[pl.ds(f * D, D), pl.ds(c * CH, CH)])


@jax.jit
def _sc_gather(tableT, xT):
    mesh = plsc.VectorSubcoreMesh(core_axis_name="c", subcore_axis_name="s")
    k = functools.partial(
        pl.kernel,
        mesh=mesh,
        out_type=jax.ShapeDtypeStruct((FCH * D, B), jnp.float32),
        name="sc_gather",
        scratch_types=[
            pltpu.VMEM((CH,), jnp.int32),
            pltpu.VMEM((D, CH), jnp.float32),
            pltpu.SemaphoreType.DMA,
        ],
        compiler_params=pltpu.CompilerParams(use_tc_tiling_on_sc=False),
    )(_sc_gather_body)
    return k(tableT, xT)


def _tc_body(hT_ref, W1T_ref, b1_ref, W2T_ref, b2_ref, Wfm_ref, WsmT_ref,
             consts_ref, out_ref):
    hT = hT_ref[...]
    dn = (((1,), (0,)), ((), ()))
    m1 = jnp.maximum(
        lax.dot_general(W1T_ref[...], hT, dn,
                        preferred_element_type=jnp.float32)
        + b1_ref[...], 0.0)
    ss = lax.dot_general(WsmT_ref[...], hT, dn,
                         preferred_element_type=jnp.float32)
    cv = consts_ref[...]
    probs = []
    for t in range(3):
        m1t = m1[t * 256:(t + 1) * 256, :]
        m2 = jnp.maximum(
            lax.dot_general(W2T_ref[t], m1t, dn,
                            preferred_element_type=jnp.float32)
            + b2_ref[t][:, None], 0.0)
        s = jnp.sum(m2 * Wfm_ref[t][:, None], axis=0, keepdims=True)
        a = ss[2 * t:2 * t + 1, :]
        c = ss[2 * t + 1:2 * t + 2, :]
        logit = a * c + c + s + cv[0, t]
        probs.append(jax.nn.sigmoid(logit))
    cvr, cf, ctr = probs
    ctcvr = cvr * ctr
    res = jnp.concatenate([cvr, cf, ctr, ctcvr], axis=0)
    out_ref[...] = jnp.clip(res, 1e-15, 1.0 - 1e-15)


@jax.jit
def _tc_dense(hT, W1T, b1all, W2T, b2all, Wfm, WsmT, consts):
    full = lambda shape: pl.BlockSpec(shape, lambda i: (0,) * len(shape))
    return pl.pallas_call(
        _tc_body,
        grid=(B // BT,),
        in_specs=[
            pl.BlockSpec((T, BT), lambda i: (0, i)),
            full((768, T)),
            full((768, 1)),
            full((3, 128, 256)),
            full((3, 128)),
            full((3, 128)),
            full((8, T)),
            full((1, 8)),
        ],
        out_specs=pl.BlockSpec((4, BT), lambda i: (0, i)),
        out_shape=jax.ShapeDtypeStruct((4, B), jnp.float32),
        compiler_params=pltpu.CompilerParams(
            dimension_semantics=("parallel",)),
    )(hT, W1T, b1all, W2T, b2all, Wfm, WsmT, consts)


def kernel(x, emb_tables,
           cvr_wc, cvr_bc, cvr_W1, cvr_b1, cvr_W2, cvr_b2, cvr_Wf, cvr_bf,
           cf_wc, cf_bc, cf_W1, cf_b1, cf_W2, cf_b2, cf_Wf, cf_bf,
           ctr_wc, ctr_bc, ctr_W1, ctr_b1, ctr_W2, ctr_b2, ctr_Wf, ctr_bf):
    xT = x.astype(jnp.int32).T

    W1T = jnp.concatenate([cvr_W1, cf_W1, ctr_W1], axis=1).T
    b1all = jnp.concatenate([cvr_b1, cf_b1, ctr_b1]).reshape(768, 1)
    W2T = jnp.stack([cvr_W2.T, cf_W2.T, ctr_W2.T])
    b2all = jnp.stack([cvr_b2, cf_b2, ctr_b2])
    Wfm = jnp.stack([cvr_Wf[T:, 0], cf_Wf[T:, 0], ctr_Wf[T:, 0]])
    zcol = jnp.zeros((T, 1), jnp.float32)
    WsmT = jnp.concatenate(
        [cvr_wc, cvr_Wf[:T], cf_wc, cf_Wf[:T], ctr_wc, ctr_Wf[:T],
         zcol, zcol], axis=1).T
    consts = jnp.stack(
        [jnp.dot(cvr_bc, cvr_Wf[:T, 0]) + cvr_bf[0],
         jnp.dot(cf_bc, cf_Wf[:T, 0]) + cf_bf[0],
         jnp.dot(ctr_bc, ctr_Wf[:T, 0]) + ctr_bf[0],
         jnp.float32(0), jnp.float32(0), jnp.float32(0),
         jnp.float32(0), jnp.float32(0)]).reshape(1, 8)

    hT_parts = [
        _sc_gather(emb_tables[k * FCH:(k + 1) * FCH].transpose(0, 2, 1),
                   xT[k * FCH:(k + 1) * FCH])
        for k in range(NKC)
    ]
    hT = jnp.concatenate(hT_parts, axis=0)

    outT = _tc_dense(hT, W1T, b1all, W2T, b2all, Wfm, WsmT, consts)
    return outT.T


# speedup vs baseline: 3.3909x; 1.0236x over previous
import functools

import jax
import jax.numpy as jnp
from jax import lax
from jax.experimental import pallas as pl
from jax.experimental.pallas import tpu as pltpu
from jax.experimental.pallas import tpu_sc as plsc

B, F, V, D = 16384, 26, 100000, 16
T = F * D
NW = 32
CH = 1024
NCB = B // CH

FPLAN = (8, 6, 6, 6)

BT = 1024


def _sc_gather_body(tpc, tableT, xT, out, idx_v, rows_v, gsem):
    wid = lax.axis_index("s") * 2 + lax.axis_index("c")

    @pl.loop(0, tpc)
    def _task(j):
        t = wid * tpc + j
        f = t >> 4
        c = t & (NCB - 1)
        pltpu.sync_copy(xT.at[f, pl.ds(c * CH, CH)], idx_v)
        copies = [
            pltpu.async_copy(tableT.at[f, d].at[idx_v], rows_v.at[d], gsem)
            for d in range(D)
        ]
        for cp in copies:
            cp.wait()
        pltpu.sync_copy(rows_v, out.at[pl.ds(f * D, D), pl.ds(c * CH, CH)])


@functools.partial(jax.jit, static_argnums=0)
def _sc_gather(fch, tableT, xT):
    mesh = plsc.VectorSubcoreMesh(core_axis_name="c", subcore_axis_name="s")
    tpc = fch * NCB // NW
    k = functools.partial(
        pl.kernel,
        mesh=mesh,
        out_type=jax.ShapeDtypeStruct((fch * D, B), jnp.float32),
        name="sc_gather",
        scratch_types=[
            pltpu.VMEM((CH,), jnp.int32),
            pltpu.VMEM((D, CH), jnp.float32),
            pltpu.SemaphoreType.DMA,
        ],
        compiler_params=pltpu.CompilerParams(use_tc_tiling_on_sc=False),
    )(functools.partial(_sc_gather_body, tpc))
    return k(tableT, xT)


def _tc_body(hT_ref, W1T_ref, b1_ref, W2T_ref, b2_ref, Wfm_ref, WsmT_ref,
             consts_ref, out_ref):
    hT = hT_ref[...]
    dn = (((1,), (0,)), ((), ()))
    m1 = jnp.maximum(
        lax.dot_general(W1T_ref[...], hT, dn,
                        preferred_element_type=jnp.float32)
        + b1_ref[...], 0.0)
    ss = lax.dot_general(WsmT_ref[...], hT, dn,
                         preferred_element_type=jnp.float32)
    cv = consts_ref[...]
    probs = []
    for t in range(3):
        m1t = m1[t * 256:(t + 1) * 256, :]
        m2 = jnp.maximum(
            lax.dot_general(W2T_ref[t], m1t, dn,
                            preferred_element_type=jnp.float32)
            + b2_ref[t][:, None], 0.0)
        s = jnp.sum(m2 * Wfm_ref[t][:, None], axis=0, keepdims=True)
        a = ss[2 * t:2 * t + 1, :]
        c = ss[2 * t + 1:2 * t + 2, :]
        logit = a * c + c + s + cv[0, t]
        probs.append(jax.nn.sigmoid(logit))
    cvr, cf, ctr = probs
    ctcvr = cvr * ctr
    res = jnp.concatenate([cvr, cf, ctr, ctcvr], axis=0)
    out_ref[...] = jnp.clip(res, 1e-15, 1.0 - 1e-15)


@jax.jit
def _tc_dense(hT, W1T, b1all, W2T, b2all, Wfm, WsmT, consts):
    full = lambda shape: pl.BlockSpec(shape, lambda i: (0,) * len(shape))
    return pl.pallas_call(
        _tc_body,
        grid=(B // BT,),
        in_specs=[
            pl.BlockSpec((T, BT), lambda i: (0, i)),
            full((768, T)),
            full((768, 1)),
            full((3, 128, 256)),
            full((3, 128)),
            full((3, 128)),
            full((8, T)),
            full((1, 8)),
        ],
        out_specs=pl.BlockSpec((4, BT), lambda i: (0, i)),
        out_shape=jax.ShapeDtypeStruct((4, B), jnp.float32),
        compiler_params=pltpu.CompilerParams(
            dimension_semantics=("parallel",)),
    )(hT, W1T, b1all, W2T, b2all, Wfm, WsmT, consts)


def kernel(x, emb_tables,
           cvr_wc, cvr_bc, cvr_W1, cvr_b1, cvr_W2, cvr_b2, cvr_Wf, cvr_bf,
           cf_wc, cf_bc, cf_W1, cf_b1, cf_W2, cf_b2, cf_Wf, cf_bf,
           ctr_wc, ctr_bc, ctr_W1, ctr_b1, ctr_W2, ctr_b2, ctr_Wf, ctr_bf):
    xT = x.astype(jnp.int32).T

    W1T = jnp.concatenate([cvr_W1, cf_W1, ctr_W1], axis=1).T
    b1all = jnp.concatenate([cvr_b1, cf_b1, ctr_b1]).reshape(768, 1)
    W2T = jnp.stack([cvr_W2.T, cf_W2.T, ctr_W2.T])
    b2all = jnp.stack([cvr_b2, cf_b2, ctr_b2])
    Wfm = jnp.stack([cvr_Wf[T:, 0], cf_Wf[T:, 0], ctr_Wf[T:, 0]])
    zcol = jnp.zeros((T, 1), jnp.float32)
    WsmT = jnp.concatenate(
        [cvr_wc, cvr_Wf[:T], cf_wc, cf_Wf[:T], ctr_wc, ctr_Wf[:T],
         zcol, zcol], axis=1).T
    consts = jnp.stack(
        [jnp.dot(cvr_bc, cvr_Wf[:T, 0]) + cvr_bf[0],
         jnp.dot(cf_bc, cf_Wf[:T, 0]) + cf_bf[0],
         jnp.dot(ctr_bc, ctr_Wf[:T, 0]) + ctr_bf[0],
         jnp.float32(0), jnp.float32(0), jnp.float32(0),
         jnp.float32(0), jnp.float32(0)]).reshape(1, 8)

    hT_parts = []
    f0 = 0
    for fch in FPLAN:
        hT_parts.append(
            _sc_gather(fch, emb_tables[f0:f0 + fch].transpose(0, 2, 1),
                       xT[f0:f0 + fch]))
        f0 += fch
    hT = jnp.concatenate(hT_parts, axis=0)

    outT = _tc_dense(hT, W1T, b1all, W2T, b2all, Wfm, WsmT, consts)
    return outT.T
